# baseline (speedup 1.0000x reference)
import jax
import jax.numpy as jnp
from jax.experimental import pallas as pl

N_ROWS = 16384
NC = 1000
VAL = 1001.0
COLS_PER_BLOCK = 4096


def _smooth_onehot_t_block(lab_ref, out_ref):
    lab = lab_ref[...]
    jrow = jax.lax.broadcasted_iota(jnp.int32, (NC, lab.shape[1]), 0)
    out_ref[...] = jnp.where(lab == jrow, VAL, 1.0)


def kernel(labels):
    c = COLS_PER_BLOCK
    lab2d = labels.astype(jnp.int32).reshape(1, N_ROWS)
    out_t = pl.pallas_call(
        _smooth_onehot_t_block,
        grid=(N_ROWS // c,),
        in_specs=[pl.BlockSpec((1, c), lambda i: (0, i))],
        out_specs=pl.BlockSpec((NC, c), lambda i: (0, i)),
        out_shape=jax.ShapeDtypeStruct((NC, N_ROWS), jnp.float32),
    )(lab2d)
    return out_t.T


# speedup vs baseline: 1.1352x; 1.1352x over previous
import jax
import jax.numpy as jnp
from jax.experimental import pallas as pl

N_ROWS = 16384
NC = 1000
VAL = 1001.0
COLS_PER_BLOCK = 1024


def _smooth_onehot_t_block(lab_ref, out_ref):
    lab = lab_ref[...]
    jrow = jax.lax.broadcasted_iota(jnp.int32, (NC, lab.shape[1]), 0)
    out_ref[...] = jnp.where(lab == jrow, VAL, 1.0)


def kernel(labels):
    c = COLS_PER_BLOCK
    lab2d = labels.astype(jnp.int32).reshape(1, N_ROWS)
    out_t = pl.pallas_call(
        _smooth_onehot_t_block,
        grid=(N_ROWS // c,),
        in_specs=[pl.BlockSpec((1, c), lambda i: (0, i))],
        out_specs=pl.BlockSpec((NC, c), lambda i: (0, i)),
        out_shape=jax.ShapeDtypeStruct((NC, N_ROWS), jnp.float32),
    )(lab2d)
    return out_t.T
